# initial kernel scaffold (unmeasured)
import jax
import jax.numpy as jnp
from jax import lax
from jax.experimental import pallas as pl
from jax.experimental.pallas import tpu as pltpu

N_Y = 2


def _ag_kv_body(K_ref, V_ref, Kf_ref, Vf_ref,
                copy_sem_k, copy_sem_v,
                send_k, recv_k, send_v, recv_v):
    my_x = lax.axis_index("x")
    my_y = lax.axis_index("y")
    my_z = lax.axis_index("z")
    neighbor = (my_x, 1 - my_y, my_z)

    barrier = pltpu.get_barrier_semaphore()
    pl.semaphore_signal(barrier, inc=1, device_id=neighbor,
                        device_id_type=pl.DeviceIdType.MESH)
    pl.semaphore_wait(barrier, 1)

    ck = pltpu.make_async_copy(K_ref, Kf_ref.at[my_y], copy_sem_k)
    cv = pltpu.make_async_copy(V_ref, Vf_ref.at[my_y], copy_sem_v)
    ck.start()
    cv.start()

    rk = pltpu.make_async_remote_copy(
        src_ref=K_ref, dst_ref=Kf_ref.at[my_y],
        send_sem=send_k, recv_sem=recv_k,
        device_id=neighbor, device_id_type=pl.DeviceIdType.MESH)
    rv = pltpu.make_async_remote_copy(
        src_ref=V_ref, dst_ref=Vf_ref.at[my_y],
        send_sem=send_v, recv_sem=recv_v,
        device_id=neighbor, device_id_type=pl.DeviceIdType.MESH)
    rk.start()
    rv.start()

    ck.wait()
    cv.wait()
    rk.wait()
    rv.wait()


def kernel(Q, K, V):
    b, s_loc, h, d = Q.shape
    scale = d ** -0.5

    kv_shape = jax.ShapeDtypeStruct((N_Y, b, s_loc, h, d), jnp.float32)
    Kf, Vf = pl.pallas_call(
        _ag_kv_body,
        out_shape=[kv_shape, kv_shape],
        in_specs=[pl.BlockSpec(memory_space=pltpu.ANY)] * 2,
        out_specs=[pl.BlockSpec(memory_space=pltpu.ANY)] * 2,
        scratch_shapes=[pltpu.SemaphoreType.DMA] * 6,
        compiler_params=pltpu.CompilerParams(collective_id=0),
    )(K, V)

    def _attn_body(Q_ref, Kf_ref, Vf_ref, o_ref):
        q = Q_ref[0, :, 0, :]
        k0 = Kf_ref[0, 0, :, 0, :]
        k1 = Kf_ref[1, 0, :, 0, :]
        v0 = Vf_ref[0, 0, :, 0, :]
        v1 = Vf_ref[1, 0, :, 0, :]
        dot_t = (((1,), (1,)), ((), ()))
        s0 = lax.dot_general(q, k0, dot_t, preferred_element_type=jnp.float32)
        s1 = lax.dot_general(q, k1, dot_t, preferred_element_type=jnp.float32)
        s = jnp.concatenate([s0, s1], axis=1) * scale
        m = jnp.max(s, axis=1, keepdims=True)
        p = jnp.exp(s - m)
        p = p / jnp.sum(p, axis=1, keepdims=True)
        dot_n = (((1,), (0,)), ((), ()))
        o = (lax.dot_general(p[:, :s_loc], v0, dot_n,
                             preferred_element_type=jnp.float32)
             + lax.dot_general(p[:, s_loc:], v1, dot_n,
                               preferred_element_type=jnp.float32))
        o_ref[0, :, 0, :] = o

    return pl.pallas_call(
        _attn_body,
        grid=(b, h),
        out_shape=jax.ShapeDtypeStruct((b, s_loc, h, d), jnp.float32),
        in_specs=[
            pl.BlockSpec((1, s_loc, 1, d), lambda i, j: (i, 0, j, 0)),
            pl.BlockSpec((N_Y, 1, s_loc, 1, d), lambda i, j: (0, i, 0, j, 0)),
            pl.BlockSpec((N_Y, 1, s_loc, 1, d), lambda i, j: (0, i, 0, j, 0)),
        ],
        out_specs=pl.BlockSpec((1, s_loc, 1, d), lambda i, j: (i, 0, j, 0)),
    )(Q, Kf, Vf)


# baseline (device time: 263898 ns/iter reference)
import jax
import jax.numpy as jnp
from jax import lax
from jax.experimental import pallas as pl
from jax.experimental.pallas import tpu as pltpu

N_Y = 2


def _ag_kv_body(K_ref, V_ref, Kf_ref, Vf_ref,
                copy_sem_k, copy_sem_v,
                send_k, recv_k, send_v, recv_v):
    my_x = lax.axis_index("x")
    my_y = lax.axis_index("y")
    my_z = lax.axis_index("z")
    neighbor = (my_x, 1 - my_y, my_z)

    barrier = pltpu.get_barrier_semaphore()
    pl.semaphore_signal(barrier, inc=1, device_id=neighbor,
                        device_id_type=pl.DeviceIdType.MESH)
    pl.semaphore_wait(barrier, 1)

    ck = pltpu.make_async_copy(K_ref, Kf_ref.at[my_y], copy_sem_k)
    cv = pltpu.make_async_copy(V_ref, Vf_ref.at[my_y], copy_sem_v)
    ck.start()
    cv.start()

    rk = pltpu.make_async_remote_copy(
        src_ref=K_ref, dst_ref=Kf_ref.at[my_y],
        send_sem=send_k, recv_sem=recv_k,
        device_id=neighbor, device_id_type=pl.DeviceIdType.MESH)
    rv = pltpu.make_async_remote_copy(
        src_ref=V_ref, dst_ref=Vf_ref.at[my_y],
        send_sem=send_v, recv_sem=recv_v,
        device_id=neighbor, device_id_type=pl.DeviceIdType.MESH)
    rk.start()
    rv.start()

    ck.wait()
    cv.wait()
    rk.wait()
    rv.wait()


def kernel(Q, K, V):
    b, s_loc, h, d = Q.shape
    scale = d ** -0.5

    kv_shape = jax.ShapeDtypeStruct((N_Y, b, s_loc, h, d), jnp.float32)
    Kf, Vf = pl.pallas_call(
        _ag_kv_body,
        out_shape=[kv_shape, kv_shape],
        in_specs=[pl.BlockSpec(memory_space=pl.ANY)] * 2,
        out_specs=[pl.BlockSpec(memory_space=pl.ANY)] * 2,
        scratch_shapes=[pltpu.SemaphoreType.DMA] * 6,
        compiler_params=pltpu.CompilerParams(collective_id=0),
    )(K, V)

    def _attn_body(Q_ref, Kf_ref, Vf_ref, o_ref):
        q = Q_ref[0]
        k0 = Kf_ref[0, 0]
        k1 = Kf_ref[1, 0]
        v0 = Vf_ref[0, 0]
        v1 = Vf_ref[1, 0]
        dot_qk = (((2,), (2,)), ((1,), (1,)))
        s0 = lax.dot_general(q, k0, dot_qk, preferred_element_type=jnp.float32)
        s1 = lax.dot_general(q, k1, dot_qk, preferred_element_type=jnp.float32)
        s = jnp.concatenate([s0, s1], axis=2) * scale
        m = jnp.max(s, axis=2, keepdims=True)
        p = jnp.exp(s - m)
        p = p / jnp.sum(p, axis=2, keepdims=True)
        dot_pv = (((2,), (0,)), ((0,), (1,)))
        o = (lax.dot_general(p[:, :, :s_loc], v0, dot_pv,
                             preferred_element_type=jnp.float32)
             + lax.dot_general(p[:, :, s_loc:], v1, dot_pv,
                               preferred_element_type=jnp.float32))
        o_ref[0] = jnp.transpose(o, (1, 0, 2))

    return pl.pallas_call(
        _attn_body,
        grid=(b,),
        out_shape=jax.ShapeDtypeStruct((b, s_loc, h, d), jnp.float32),
        in_specs=[
            pl.BlockSpec((1, s_loc, h, d), lambda i: (i, 0, 0, 0)),
            pl.BlockSpec((N_Y, 1, s_loc, h, d), lambda i: (0, i, 0, 0, 0)),
            pl.BlockSpec((N_Y, 1, s_loc, h, d), lambda i: (0, i, 0, 0, 0)),
        ],
        out_specs=pl.BlockSpec((1, s_loc, h, d), lambda i: (i, 0, 0, 0)),
    )(Q, Kf, Vf)
